# Initial kernel scaffold; baseline (speedup 1.0000x reference)
import jax
import jax.numpy as jnp
from jax.experimental import pallas as pl


def kernel(des, tweet, num_prop, cat_prop, edge_index, edge_type, W_des, b_des, W_tweet, b_tweet, W_num, b_num, W_cat, b_cat, W_in, b_in, rgcn_w, rgcn_root, rgcn_bias, W_out1, b_out1, W_out2, b_out2):
    raise NotImplementedError("write your pallas kernel here")


# speedup vs baseline: 4.6817x; 4.6817x over previous
import functools

import jax
import jax.numpy as jnp
from jax import lax
from jax.experimental import pallas as pl
from jax.experimental.pallas import tpu as pltpu
from jax.experimental.pallas import tpu_sc as plsc

N = 10000
E = 320000
D = 128

BN = 1000
NB = N // BN

CHUNK = 128
CPT = 160
IB = 16
NBLK = CPT // IB
E_PAD = CPT * 16 * CHUNK
ACC = 20096
RPT = ACC // 16
PAD_ROW = 20090
CW = 8


def _leaky(v):
    return jnp.where(v >= 0, v, 0.01 * v)


def _idx_body(src_ref, typ_ref, dst_ref, g0_ref, g1_ref, s_ref):
    src = src_ref[...]
    typ = typ_ref[...]
    g0 = 2 * (src + typ * N)
    g0_ref[...] = g0
    g1_ref[...] = g0 + 1
    s_ref[...] = dst_ref[...] + typ * N


def _make_indices(src_p, typ_p, dst_p):
    rows = E_PAD // 128
    shp = jax.ShapeDtypeStruct((rows, 128), jnp.int32)
    return pl.pallas_call(_idx_body, out_shape=[shp, shp, shp])(
        src_p.reshape(rows, 128), typ_p.reshape(rows, 128),
        dst_p.reshape(rows, 128))


def _feat_body(des_ref, tweet_ref, num_ref, cat_ref,
               Wd_ref, bd_ref, Wt_ref, bt_ref, Wn_ref, bn_ref,
               Wc_ref, bc_ref, Win_ref, bin_ref, x_ref):
    d = _leaky(jnp.dot(des_ref[...], Wd_ref[...],
                       preferred_element_type=jnp.float32) + bd_ref[...])
    t = _leaky(jnp.dot(tweet_ref[...], Wt_ref[...],
                       preferred_element_type=jnp.float32) + bt_ref[...])
    n = _leaky(jnp.dot(num_ref[...], Wn_ref[...],
                       preferred_element_type=jnp.float32) + bn_ref[...])
    c = _leaky(jnp.dot(cat_ref[...], Wc_ref[...],
                       preferred_element_type=jnp.float32) + bc_ref[...])
    Win = Win_ref[...]
    acc = (jnp.dot(d, Win[0:32, :], preferred_element_type=jnp.float32)
           + jnp.dot(t, Win[32:64, :], preferred_element_type=jnp.float32)
           + jnp.dot(n, Win[64:96, :], preferred_element_type=jnp.float32)
           + jnp.dot(c, Win[96:128, :], preferred_element_type=jnp.float32))
    x_ref[...] = _leaky(acc + bin_ref[...])


def _features(des, tweet, num_prop, cat_prop, W_des, b_des, W_tweet, b_tweet,
              W_num, b_num, W_cat, b_cat, W_in, b_in):
    full = lambda shape: pl.BlockSpec(shape, lambda i: tuple(0 for _ in shape))
    return pl.pallas_call(
        _feat_body,
        grid=(NB,),
        in_specs=[
            pl.BlockSpec((BN, 768), lambda i: (i, 0)),
            pl.BlockSpec((BN, 768), lambda i: (i, 0)),
            pl.BlockSpec((BN, 5), lambda i: (i, 0)),
            pl.BlockSpec((BN, 3), lambda i: (i, 0)),
            full((768, 32)), full((1, 32)),
            full((768, 32)), full((1, 32)),
            full((5, 32)), full((1, 32)),
            full((3, 32)), full((1, 32)),
            full((128, 128)), full((1, 128)),
        ],
        out_specs=pl.BlockSpec((BN, 128), lambda i: (i, 0)),
        out_shape=jax.ShapeDtypeStruct((N, 128), jnp.float32),
    )(des, tweet, num_prop, cat_prop,
      W_des, b_des.reshape(1, 32), W_tweet, b_tweet.reshape(1, 32),
      W_num, b_num.reshape(1, 32), W_cat, b_cat.reshape(1, 32),
      W_in, b_in.reshape(1, 128))


def _h_body(x_ref, w_ref, h_ref):
    h_ref[...] = jnp.dot(x_ref[...], w_ref[0],
                         preferred_element_type=jnp.float32)


def _transform(x, rgcn_w):
    h = pl.pallas_call(
        _h_body,
        grid=(2, NB),
        in_specs=[
            pl.BlockSpec((BN, 128), lambda r, i: (i, 0)),
            pl.BlockSpec((1, 128, 128), lambda r, i: (r, 0, 0)),
        ],
        out_specs=pl.BlockSpec((BN, 128), lambda r, i: (r * NB + i, 0)),
        out_shape=jax.ShapeDtypeStruct((2 * N, 128), jnp.float32),
    )(x, rgcn_w)
    return h.reshape(4 * N, 64)


def _sc_pass(do_counts):
    mesh = plsc.VectorSubcoreMesh(core_axis_name="c", subcore_axis_name="s")
    out_type = [jax.ShapeDtypeStruct((2, ACC, 64), jnp.float32)]
    scratch = [
        pltpu.VMEM((IB, CHUNK), jnp.int32),
        pltpu.VMEM((IB, CHUNK), jnp.int32),
        pltpu.VMEM((CHUNK, 64), jnp.float32),
        pltpu.VMEM_SHARED((ACC, 64), jnp.float32),
        pltpu.SemaphoreType.DMA,
    ]
    if do_counts:
        out_type.append(jax.ShapeDtypeStruct((ACC, CW), jnp.float32))
        scratch += [
            pltpu.VMEM((CHUNK, CW), jnp.float32),
            pltpu.VMEM_SHARED((ACC, CW), jnp.float32),
        ]

    def body(h2, gidx, sidx, zer64, *rest):
        if do_counts:
            (zerc, ones, s2_out, c_out,
             gi_v, si_v, rows_v, acc, sem, ones_v, cacc) = rest
        else:
            s2_out, gi_v, si_v, rows_v, acc, sem = rest
        c = lax.axis_index("c")
        s = lax.axis_index("s")
        base = s * RPT
        pltpu.sync_copy(zer64, acc.at[pl.ds(base, RPT)])
        if do_counts:
            pltpu.sync_copy(zerc, cacc.at[pl.ds(base, RPT)])
            pltpu.sync_copy(ones, ones_v)
        plsc.subcore_barrier()

        def block(b, carry):
            pltpu.sync_copy(gidx.at[c, s, b], gi_v)
            pltpu.sync_copy(sidx.at[s, b], si_v)

            def step(k, inner):
                pltpu.async_copy(h2.at[gi_v.at[k]], rows_v, sem).wait()
                pltpu.sync_copy(rows_v, acc.at[si_v.at[k]], add=True)
                if do_counts:
                    @pl.when(c == 0)
                    def _():
                        pltpu.sync_copy(ones_v, cacc.at[si_v.at[k]], add=True)
                return inner

            return lax.fori_loop(0, IB, step, carry)

        lax.fori_loop(0, NBLK, block, 0)
        plsc.subcore_barrier()
        pltpu.sync_copy(acc.at[pl.ds(base, RPT)],
                        s2_out.at[c].at[pl.ds(base, RPT)])
        if do_counts:
            @pl.when(c == 0)
            def _():
                pltpu.sync_copy(cacc.at[pl.ds(base, RPT)],
                                c_out.at[pl.ds(base, RPT)])

    return functools.partial(
        pl.kernel, mesh=mesh, out_type=out_type, scratch_types=scratch,
        compiler_params=pltpu.CompilerParams(use_tc_tiling_on_sc=False),
    )(body)


_sc_messages_counts = _sc_pass(True)
_sc_messages = _sc_pass(False)


def _combine_core(x_ref, s0l, s0r, s1l, s1r, c0, c1, root_ref, bias_ref):
    inv0 = 1.0 / jnp.maximum(c0[:, 0:1], 1.0)
    inv1 = 1.0 / jnp.maximum(c1[:, 0:1], 1.0)
    m0 = jnp.concatenate([s0l[0], s0r[0]], axis=1) * inv0
    m1 = jnp.concatenate([s1l[0], s1r[0]], axis=1) * inv1
    base = jnp.dot(x_ref[...], root_ref[...],
                   preferred_element_type=jnp.float32) + bias_ref[...]
    return base + m0 + m1


def _comb_body(x_ref, s0l, s0r, s1l, s1r, c0, c1, root_ref, bias_ref, out_ref):
    out_ref[...] = _combine_core(x_ref, s0l[...], s0r[...], s1l[...], s1r[...],
                                 c0[...], c1[...], root_ref, bias_ref)


def _head_body(x_ref, s0l, s0r, s1l, s1r, c0, c1, root_ref, bias_ref,
               W1_ref, b1_ref, W2_ref, b2_ref, out_ref):
    x3 = _combine_core(x_ref, s0l[...], s0r[...], s1l[...], s1r[...],
                       c0[...], c1[...], root_ref, bias_ref)
    h = _leaky(jnp.dot(x3, W1_ref[...],
                       preferred_element_type=jnp.float32) + b1_ref[...])
    out_ref[...] = jnp.dot(h, W2_ref[...],
                           preferred_element_type=jnp.float32) + b2_ref[...]


def _rgcn_specs():
    full = lambda shape: pl.BlockSpec(shape, lambda i: tuple(0 for _ in shape))
    return [
        pl.BlockSpec((BN, 128), lambda i: (i, 0)),
        pl.BlockSpec((1, BN, 64), lambda i: (0, i, 0)),
        pl.BlockSpec((1, BN, 64), lambda i: (1, i, 0)),
        pl.BlockSpec((1, BN, 64), lambda i: (0, NB + i, 0)),
        pl.BlockSpec((1, BN, 64), lambda i: (1, NB + i, 0)),
        pl.BlockSpec((BN, CW), lambda i: (i, 0)),
        pl.BlockSpec((BN, CW), lambda i: (NB + i, 0)),
        full((128, 128)), full((1, 128)),
    ]


def _combine(x, s2, cnt, root, bias):
    return pl.pallas_call(
        _comb_body,
        grid=(NB,),
        in_specs=_rgcn_specs(),
        out_specs=pl.BlockSpec((BN, 128), lambda i: (i, 0)),
        out_shape=jax.ShapeDtypeStruct((N, 128), jnp.float32),
    )(x, s2, s2, s2, s2, cnt, cnt, root, bias.reshape(1, 128))


def _head(x, s2, cnt, root, bias, W1, b1, W2p, b2p):
    full = lambda shape: pl.BlockSpec(shape, lambda i: tuple(0 for _ in shape))
    return pl.pallas_call(
        _head_body,
        grid=(NB,),
        in_specs=_rgcn_specs() + [full((128, 128)), full((1, 128)),
                                  full((128, 128)), full((1, 128))],
        out_specs=pl.BlockSpec((BN, 128), lambda i: (i, 0)),
        out_shape=jax.ShapeDtypeStruct((N, 128), jnp.float32),
    )(x, s2, s2, s2, s2, cnt, cnt, root, bias.reshape(1, 128),
      W1, b1.reshape(1, 128), W2p, b2p)


def kernel(des, tweet, num_prop, cat_prop, edge_index, edge_type,
           W_des, b_des, W_tweet, b_tweet, W_num, b_num, W_cat, b_cat,
           W_in, b_in, rgcn_w, rgcn_root, rgcn_bias,
           W_out1, b_out1, W_out2, b_out2):
    pad = E_PAD - E
    src_p = jnp.concatenate([edge_index[0], jnp.zeros((pad,), jnp.int32)])
    dst_p = jnp.concatenate([edge_index[1],
                             jnp.full((pad,), PAD_ROW, jnp.int32)])
    typ_p = jnp.concatenate([edge_type, jnp.zeros((pad,), jnp.int32)])

    g0, g1, sx = _make_indices(src_p, typ_p, dst_p)
    gidx = jnp.stack([g0, g1]).reshape(2, 16, NBLK, IB, CHUNK)
    sidx = sx.reshape(16, NBLK, IB, CHUNK)

    zer64 = jnp.zeros((RPT, 64), jnp.float32)
    zerc = jnp.zeros((RPT, CW), jnp.float32)
    ones = jnp.ones((CHUNK, CW), jnp.float32)

    x = _features(des, tweet, num_prop, cat_prop, W_des, b_des,
                  W_tweet, b_tweet, W_num, b_num, W_cat, b_cat, W_in, b_in)

    h4 = _transform(x, rgcn_w)
    s2_1, cnt = _sc_messages_counts(h4, gidx, sidx, zer64, zerc, ones)
    x2 = _combine(x, s2_1, cnt, rgcn_root, rgcn_bias)

    h4b = _transform(x2, rgcn_w)
    (s2_2,) = _sc_messages(h4b, gidx, sidx, zer64)

    W2p = jnp.zeros((128, 128), jnp.float32).at[:, :2].set(W_out2)
    b2p = jnp.zeros((1, 128), jnp.float32).at[0, :2].set(b_out2)
    out = _head(x2, s2_2, cnt, rgcn_root, rgcn_bias, W_out1, b_out1, W2p, b2p)
    return out[:, :2]
